# baseline (speedup 1.0000x reference)
import functools

import jax
import jax.numpy as jnp
from jax import lax
from jax.experimental import pallas as pl
from jax.experimental.pallas import tpu as pltpu
from jax.experimental.pallas import tpu_sc as plsc

_D = 64
_J = 200
_BB = 4096
_NW = 32
_BLK = _BB // _NW
_JB = 8
_NJB = _J // _JB
_L = 16


@functools.partial(
    pl.kernel,
    out_type=jax.ShapeDtypeStruct((_J * _D, _BB), jnp.float32),
    mesh=plsc.VectorSubcoreMesh(core_axis_name="c", subcore_axis_name="s"),
    compiler_params=pltpu.CompilerParams(needs_layout_passes=False),
    scratch_types=[
        pltpu.VMEM((_JB, _BLK), jnp.int32),
        pltpu.VMEM((_JB, _BLK), jnp.int32),
        pltpu.VMEM((_BLK, 128), jnp.float32),
        pltpu.VMEM((_D, _BLK), jnp.float32),
        pltpu.SemaphoreType.DMA,
    ],
)
def _embed_gather(xt_hbm, table_hbm, out_hbm, idx_v, half_v, rows_v, outt_v, sem):
    wid = lax.axis_index("s") * 2 + lax.axis_index("c")
    col0 = pl.multiple_of(wid * _BLK, _BLK)

    def step(ja, carry):
        j0 = pl.multiple_of(ja * _JB, _JB)
        pltpu.sync_copy(xt_hbm.at[pl.ds(j0, _JB), pl.ds(col0, _BLK)], idx_v)
        for jj in range(_JB):
            for i in range(_BLK // _L):
                v = idx_v[jj, pl.ds(i * _L, _L)]
                half_v[jj, pl.ds(i * _L, _L)] = lax.shift_right_logical(v, 1)

        def col_body(jj, carry2):
            pltpu.async_copy(
                table_hbm.at[half_v.at[jj]], rows_v, sem
            ).wait()
            rvecs, cvecs = [], []
            for i in range(_BLK // _L):
                rvecs.append(lax.iota(jnp.int32, _L) + (i * _L))
                par = lax.shift_left(
                    lax.bitwise_and(idx_v[jj, pl.ds(i * _L, _L)], 1), 6
                )
                cvecs.append(par)
            for d in range(_D):
                for i in range(_BLK // _L):
                    vals = plsc.load_gather(rows_v, [rvecs[i], cvecs[i] + d])
                    outt_v[d, pl.ds(i * _L, _L)] = vals
            row0 = pl.multiple_of((j0 + jj) * _D, _D)
            pltpu.sync_copy(
                outt_v, out_hbm.at[pl.ds(row0, _D), pl.ds(col0, _BLK)]
            )
            return carry2

        lax.fori_loop(0, _JB, col_body, 0)
        return carry

    lax.fori_loop(0, _NJB, step, 0)


def kernel(x, table):
    xt = x.T
    table2 = table.reshape(500_000, 128)
    out = _embed_gather(xt, table2)
    return out.reshape(_J, _D, _BB).transpose(2, 0, 1)
